# Initial kernel scaffold; baseline (speedup 1.0000x reference)
import jax
import jax.numpy as jnp
from jax.experimental import pallas as pl


def kernel(x, wts, vecs):
    raise NotImplementedError("write your pallas kernel here")


# speedup vs baseline: 1.7540x; 1.7540x over previous
import jax
import jax.numpy as jnp
from jax import lax
from jax.experimental import pallas as pl
from jax.experimental.pallas import tpu as pltpu

_EPS = 1e-6
_D_IN = 32
_D_H = 64
_D_OUT = 8
_TM = 512


def _ln(h, d, g, b):
    s = jnp.sum(h, axis=-1, keepdims=True)
    ss = jnp.sum(h * h, axis=-1, keepdims=True)
    mean = s * (1.0 / d)
    var = (ss - s * mean) * (1.0 / (d - 1))
    inv = pl.reciprocal(jnp.sqrt(var) + _EPS, approx=True)
    return g * (h - mean) * inv + b


def _mlp_kernel(x_ref, w_ref, v_ref, o_ref):
    f32 = jnp.float32
    x = x_ref[...]

    xn = _ln(x, _D_IN, v_ref[0:1, :_D_IN], v_ref[1:2, :_D_IN])

    h = jnp.dot(xn, w_ref[0, :_D_IN, :], preferred_element_type=f32)
    h = h + v_ref[2:3, :]
    h = _ln(h, _D_H, v_ref[3:4, :], v_ref[4:5, :])
    h = jnp.maximum(h, 0.0)

    h = jnp.dot(h, w_ref[1], preferred_element_type=f32)
    h = h + v_ref[5:6, :]
    h = _ln(h, _D_H, v_ref[6:7, :], v_ref[7:8, :])
    h = jnp.maximum(h, 0.0)

    h = jnp.dot(h, w_ref[2], preferred_element_type=f32)
    h = h + v_ref[8:9, :]
    col = lax.broadcasted_iota(jnp.int32, h.shape, 1)
    hm = jnp.where(col < _D_OUT, h, jnp.float32(-jnp.inf))
    m = jnp.max(hm, axis=-1, keepdims=True)
    lse = m + jnp.log(jnp.sum(jnp.exp(hm - m), axis=-1, keepdims=True))
    z = (h - lse)[:, :_D_OUT]

    ocol = lax.broadcasted_iota(jnp.int32, z.shape, 1)
    r = jnp.round(z * 1000.0) / 1000.0
    cond = (ocol == 0) & (r >= 0.5) & (r <= 1.0)
    o_ref[...] = jnp.where(cond, jnp.float32(0.8), z)


def kernel(x, wts, vecs):
    B, D = x.shape
    tm = _TM if B % _TM == 0 else max(8, B)
    if B % tm:
        bp = ((B + tm - 1) // tm) * tm
        x = jnp.zeros((bp, D), jnp.float32).at[:B].set(x)
    bp = x.shape[0]
    out = pl.pallas_call(
        _mlp_kernel,
        out_shape=jax.ShapeDtypeStruct((bp, _D_OUT), jnp.float32),
        grid=(bp // tm,),
        in_specs=[
            pl.BlockSpec((tm, D), lambda i: (i, 0)),
            pl.BlockSpec((3, 128, 128), lambda i: (0, 0, 0)),
            pl.BlockSpec((16, 128), lambda i: (0, 0)),
        ],
        out_specs=pl.BlockSpec((tm, _D_OUT), lambda i: (i, 0)),
        compiler_params=pltpu.CompilerParams(
            dimension_semantics=("parallel",)),
    )(x, wts, vecs)
    return out[:B]


# speedup vs baseline: 4.1516x; 2.3670x over previous
import jax
import jax.numpy as jnp
from jax import lax
from jax.experimental import pallas as pl
from jax.experimental.pallas import tpu as pltpu

_EPS = 1e-6
_D_IN = 32
_D_H = 64
_D_OUT = 8
_TM = 4096


def _sln(h, d, g, b):
    s = jnp.sum(h, axis=0, keepdims=True)
    ss = jnp.sum(h * h, axis=0, keepdims=True)
    mean = s * (1.0 / d)
    var = (ss - s * mean) * (1.0 / (d - 1))
    inv = pl.reciprocal(jnp.sqrt(var) + _EPS, approx=True)
    return g * (h - mean) * inv + b


def _mlp_kernel(x_ref, w_ref, v_ref, o_ref):
    f32 = jnp.float32
    xt = x_ref[...].T

    xn = _sln(xt, _D_IN, v_ref[0, :_D_IN, :], v_ref[1, :_D_IN, :])

    h = jnp.dot(w_ref[0, :_D_H, :_D_IN], xn, preferred_element_type=f32)
    h = h + v_ref[2, :_D_H, :]
    h = _sln(h, _D_H, v_ref[3, :_D_H, :], v_ref[4, :_D_H, :])
    h = jnp.maximum(h, 0.0)

    h = jnp.dot(w_ref[1, :_D_H, :_D_H], h, preferred_element_type=f32)
    h = h + v_ref[5, :_D_H, :]
    h = _sln(h, _D_H, v_ref[6, :_D_H, :], v_ref[7, :_D_H, :])
    h = jnp.maximum(h, 0.0)

    z = jnp.dot(w_ref[2, :_D_OUT, :_D_H], h, preferred_element_type=f32)
    z = z + v_ref[8, :_D_OUT, :]
    m = jnp.max(z, axis=0, keepdims=True)
    lse = m + jnp.log(jnp.sum(jnp.exp(z - m), axis=0, keepdims=True))
    z = z - lse

    row = lax.broadcasted_iota(jnp.int32, z.shape, 0)
    r = jnp.round(z * 1000.0) / 1000.0
    cond = (row == 0) & (r >= 0.5) & (r <= 1.0)
    z = jnp.where(cond, jnp.float32(0.8), z)

    o_ref[...] = z.T


def kernel(x, wts, vecs):
    B, D = x.shape
    tm = _TM if B % _TM == 0 else max(8, B)
    if B % tm:
        bp = ((B + tm - 1) // tm) * tm
        x = jnp.zeros((bp, D), jnp.float32).at[:B].set(x)
    bp = x.shape[0]

    wts_t = wts.transpose(0, 2, 1)
    v_bc = jnp.broadcast_to(vecs[:9, :_D_H, None], (9, _D_H, tm))

    out = pl.pallas_call(
        _mlp_kernel,
        out_shape=jax.ShapeDtypeStruct((bp, _D_OUT), jnp.float32),
        grid=(bp // tm,),
        in_specs=[
            pl.BlockSpec((tm, D), lambda i: (i, 0)),
            pl.BlockSpec((3, 128, 128), lambda i: (0, 0, 0)),
            pl.BlockSpec((9, _D_H, tm), lambda i: (0, 0, 0)),
        ],
        out_specs=pl.BlockSpec((tm, _D_OUT), lambda i: (i, 0)),
        compiler_params=pltpu.CompilerParams(
            dimension_semantics=("parallel",)),
    )(x, wts_t, v_bc)
    return out[:B]


# speedup vs baseline: 6.2197x; 1.4981x over previous
import jax
import jax.numpy as jnp
from jax import lax
from jax.experimental import pallas as pl
from jax.experimental.pallas import tpu as pltpu

_EPS = 1e-6
_D_IN = 32
_D_H = 64
_D_OUT = 8
_TM = 4096


def _sln(h, d, g, b):
    s = jnp.sum(h, axis=0, keepdims=True)
    ss = jnp.sum(h * h, axis=0, keepdims=True)
    mean = s * (1.0 / d)
    var = (ss - s * mean) * (1.0 / (d - 1))
    inv = pl.reciprocal(jnp.sqrt(var) + _EPS, approx=True)
    return g * (h - mean) * inv + b


def _mlp_kernel(x_ref, w_ref, v_ref, o_ref):
    f32 = jnp.float32
    xt = x_ref[...].T

    xn = _sln(xt, _D_IN, v_ref[0, :_D_IN, :], v_ref[1, :_D_IN, :])

    h = jnp.dot(w_ref[0, :_D_H, :_D_IN], xn, preferred_element_type=f32)
    h = h + v_ref[2, :_D_H, :]
    h = _sln(h, _D_H, v_ref[3, :_D_H, :], v_ref[4, :_D_H, :])
    h = jnp.maximum(h, 0.0)

    h = jnp.dot(w_ref[1, :_D_H, :_D_H], h, preferred_element_type=f32)
    h = h + v_ref[5, :_D_H, :]
    h = _sln(h, _D_H, v_ref[6, :_D_H, :], v_ref[7, :_D_H, :])
    h = jnp.maximum(h, 0.0)

    z = jnp.dot(w_ref[2, :_D_OUT, :_D_H], h, preferred_element_type=f32)
    z = z + v_ref[8, :_D_OUT, :]
    m = jnp.max(z, axis=0, keepdims=True)
    lse = m + jnp.log(jnp.sum(jnp.exp(z - m), axis=0, keepdims=True))
    z = z - lse

    row = lax.broadcasted_iota(jnp.int32, z.shape, 0)
    r = jnp.round(z * 1000.0) / 1000.0
    cond = (row == 0) & (r >= 0.5) & (r <= 1.0)
    o_ref[...] = jnp.where(cond, jnp.float32(0.8), z)


def kernel(x, wts, vecs):
    B, D = x.shape
    tm = _TM if B % _TM == 0 else max(8, B)
    if B % tm:
        bp = ((B + tm - 1) // tm) * tm
        x = jnp.zeros((bp, D), jnp.float32).at[:B].set(x)
    bp = x.shape[0]

    wts_t = wts.transpose(0, 2, 1)
    v_bc = jnp.broadcast_to(vecs[:9, :_D_H, None], (9, _D_H, tm))

    out = pl.pallas_call(
        _mlp_kernel,
        out_shape=jax.ShapeDtypeStruct((_D_OUT, bp), jnp.float32),
        grid=(bp // tm,),
        in_specs=[
            pl.BlockSpec((tm, D), lambda i: (i, 0)),
            pl.BlockSpec((3, 128, 128), lambda i: (0, 0, 0)),
            pl.BlockSpec((9, _D_H, tm), lambda i: (0, 0, 0)),
        ],
        out_specs=pl.BlockSpec((_D_OUT, tm), lambda i: (0, i)),
        compiler_params=pltpu.CompilerParams(
            dimension_semantics=("parallel",)),
    )(x, wts_t, v_bc)
    return out.T[:B]
